# baseline (speedup 1.0000x reference)
import jax
import jax.numpy as jnp
from jax.experimental import pallas as pl
from jax.experimental.pallas import tpu as pltpu


def _w3(w):
    w_hi = w.astype(jnp.bfloat16)
    w_lo = (w - w_hi.astype(jnp.float32)).astype(jnp.bfloat16)
    return jnp.concatenate([w_hi, w_lo, w_hi], axis=0)


def _x3(x):
    x_hi = x.astype(jnp.bfloat16)
    x_lo = (x - x_hi.astype(jnp.float32)).astype(jnp.bfloat16)
    return jnp.concatenate([x_hi, x_hi, x_lo], axis=1)


def _dot3(x, w3):
    return jnp.dot(_x3(x), w3, preferred_element_type=jnp.float32)


def _seq_gru_kernel(hid0_ref, xm_ref, fm_ref, x_ref, m_ref,
                    whw_ref, bhw_ref, w2_ref, b2_ref,
                    wih_ref, bih_ref, wp_ref,
                    hout_ref, loss_ref, lpre_ref,
                    h_scr):
    B, H = hid0_ref.shape
    D = x_ref.shape[1]
    T = xm_ref.shape[1]

    h_scr[...] = hid0_ref[...]
    whw = whw_ref[...]
    bhw = bhw_ref[...]
    w2 = w2_ref[...]
    b2 = b2_ref[...]
    wih = wih_ref[...]
    bih = bih_ref[...]
    wp = wp_ref[...]
    xm = xm_ref[...]
    fm = fm_ref[...]
    lane_iota = jax.lax.broadcasted_iota(jnp.int32, (B, D), 1).astype(jnp.float32)
    col_iota = jax.lax.broadcasted_iota(jnp.int32, (B, T), 1)

    def step(t, hidden):
        hw = _dot3(hidden, whw) + bhw
        a = jnp.maximum(hw[:, :H], 0.0)
        gh = hw[:, H:]
        p = _dot3(a, w2) + b2
        mean = p[:, :D]
        logvar = p[:, D:]

        colf = (col_iota == t).astype(jnp.float32)
        xs = jnp.sum(xm * colf, axis=1, keepdims=True)
        fs = jnp.sum(fm * colf, axis=1, keepdims=True)
        onehot = (lane_iota == fs).astype(jnp.float32)
        mean_f = jnp.sum(mean * onehot, axis=1, keepdims=True)
        logvar_f = jnp.sum(logvar * onehot, axis=1, keepdims=True)
        sigma = jnp.exp(0.5 * logvar_f)
        err = (xs - mean_f) / sigma
        dloss = 0.5 * jnp.sum(err * err + logvar_f)

        lhs = jnp.concatenate(
            [onehot * xs, onehot * mean_f, onehot * logvar_f, onehot * err,
             onehot], axis=1)
        gru_in = jnp.maximum(_dot3(lhs, wp), 0.0)
        gi = _dot3(gru_in, wih) + bih
        r = jax.nn.sigmoid(gi[:, :H] + gh[:, :H])
        z = jax.nn.sigmoid(gi[:, H:2 * H] + gh[:, H:2 * H])
        n = jnp.tanh(gi[:, 2 * H:] + r * gh[:, 2 * H:])
        h_new = (1.0 - z) * n + z * hidden
        return h_new, dloss, mean, logvar

    h_new, loss, mean, logvar = step(0, h_scr[...])
    h_scr[...] = h_new
    sigma0 = jnp.exp(0.5 * logvar)
    e0 = (x_ref[...] - mean) / sigma0
    lpre_ref[...] = 0.5 * ((e0 * e0 + logvar) * m_ref[...])

    def body(t, loss):
        h_new, dloss, _, _ = step(t, h_scr[...])
        h_scr[...] = h_new
        return loss + dloss

    loss = jax.lax.fori_loop(1, T, body, loss, unroll=4)
    loss_ref[0, 0] = loss
    hout_ref[...] = h_scr[...]


def kernel(h, X_obs_data, F_obs_data, i_obs, X, M, W1, b1, W2, b2,
           W_ih, W_hh, b_ih, b_hh, w_prep, bias_prep):
    B = i_obs.shape[0]
    T = X_obs_data.shape[0] // B
    D = X.shape[1]
    H = h.shape[1]
    P = w_prep.shape[2]

    hidden0 = h[i_obs]
    Xm = X_obs_data.reshape(T, B).T
    Fm = F_obs_data.reshape(T, B).astype(jnp.float32).T
    Whw = jnp.concatenate([W1, W_hh.T], axis=1)
    bhw = jnp.concatenate([b1, b_hh]).reshape(1, -1)
    Wp = jnp.concatenate(
        [jnp.transpose(w_prep, (1, 0, 2)).reshape(4 * D, P), bias_prep],
        axis=0)

    out_shapes = (
        jax.ShapeDtypeStruct((B, H), jnp.float32),
        jax.ShapeDtypeStruct((1, 1), jnp.float32),
        jax.ShapeDtypeStruct((B, D), jnp.float32),
    )
    hout, loss, lpre = pl.pallas_call(
        _seq_gru_kernel,
        out_shape=out_shapes,
        out_specs=(
            pl.BlockSpec(memory_space=pltpu.VMEM),
            pl.BlockSpec(memory_space=pltpu.SMEM),
            pl.BlockSpec(memory_space=pltpu.VMEM),
        ),
        scratch_shapes=[pltpu.VMEM((B, H), jnp.float32)],
    )(hidden0, Xm, Fm, X, M,
      _w3(Whw), bhw, _w3(W2), b2.reshape(1, -1),
      _w3(W_ih.T), b_ih.reshape(1, -1), _w3(Wp))

    h2 = h.at[i_obs].set(hout)
    return (h2, loss[0, 0], lpre)


# speedup vs baseline: 2.0847x; 2.0847x over previous
import jax
import jax.numpy as jnp
from jax.experimental import pallas as pl
from jax.experimental.pallas import tpu as pltpu


def _dot(x, w):
    return jnp.dot(x, w, preferred_element_type=jnp.float32)


def _seq_gru_kernel(hid0_ref, xm_ref, fm_ref, x_ref, m_ref,
                    whw_ref, bhw_ref, w2_ref, b2_ref,
                    wih_ref, bih_ref, wp_ref,
                    hout_ref, loss_ref, lpre_ref,
                    h_scr):
    B, H = hid0_ref.shape
    D = x_ref.shape[1]
    T = xm_ref.shape[1]

    h_scr[...] = hid0_ref[...]
    whw = whw_ref[...]
    bhw = bhw_ref[...]
    w2 = w2_ref[...]
    b2 = b2_ref[...]
    wih = wih_ref[...]
    bih = bih_ref[...]
    wp = wp_ref[...]
    xm = xm_ref[...]
    fm = fm_ref[...]
    lane_iota = jax.lax.broadcasted_iota(jnp.int32, (B, D), 1).astype(jnp.float32)
    col_iota = jax.lax.broadcasted_iota(jnp.int32, (B, T), 1)

    def step(t, hidden):
        hw = _dot(hidden, whw) + bhw
        a = jnp.maximum(hw[:, :H], 0.0)
        gh = hw[:, H:]
        p = _dot(a, w2) + b2
        mean = p[:, :D]
        logvar = p[:, D:]

        colf = (col_iota == t).astype(jnp.float32)
        xs = jnp.sum(xm * colf, axis=1, keepdims=True)
        fs = jnp.sum(fm * colf, axis=1, keepdims=True)
        onehot = (lane_iota == fs).astype(jnp.float32)
        mean_f = jnp.sum(mean * onehot, axis=1, keepdims=True)
        logvar_f = jnp.sum(logvar * onehot, axis=1, keepdims=True)
        sigma = jnp.exp(0.5 * logvar_f)
        err = (xs - mean_f) / sigma
        dloss = 0.5 * jnp.sum(err * err + logvar_f)

        lhs = jnp.concatenate(
            [onehot * xs, onehot * mean_f, onehot * logvar_f, onehot * err,
             onehot], axis=1)
        gru_in = jnp.maximum(_dot(lhs, wp), 0.0)
        gi = _dot(gru_in, wih) + bih
        r = jax.nn.sigmoid(gi[:, :H] + gh[:, :H])
        z = jax.nn.sigmoid(gi[:, H:2 * H] + gh[:, H:2 * H])
        n = jnp.tanh(gi[:, 2 * H:] + r * gh[:, 2 * H:])
        h_new = (1.0 - z) * n + z * hidden
        return h_new, dloss, mean, logvar

    h_new, loss, mean, logvar = step(0, h_scr[...])
    h_scr[...] = h_new
    sigma0 = jnp.exp(0.5 * logvar)
    e0 = (x_ref[...] - mean) / sigma0
    lpre_ref[...] = 0.5 * ((e0 * e0 + logvar) * m_ref[...])

    def body(t, loss):
        h_new, dloss, _, _ = step(t, h_scr[...])
        h_scr[...] = h_new
        return loss + dloss

    loss = jax.lax.fori_loop(1, T, body, loss, unroll=4)
    loss_ref[0, 0] = loss
    hout_ref[...] = h_scr[...]


def kernel(h, X_obs_data, F_obs_data, i_obs, X, M, W1, b1, W2, b2,
           W_ih, W_hh, b_ih, b_hh, w_prep, bias_prep):
    B = i_obs.shape[0]
    T = X_obs_data.shape[0] // B
    D = X.shape[1]
    H = h.shape[1]
    P = w_prep.shape[2]

    hidden0 = h[i_obs]
    Xm = X_obs_data.reshape(T, B).T
    Fm = F_obs_data.reshape(T, B).astype(jnp.float32).T
    Whw = jnp.concatenate([W1, W_hh.T], axis=1)
    bhw = jnp.concatenate([b1, b_hh]).reshape(1, -1)
    Wp = jnp.concatenate(
        [jnp.transpose(w_prep, (1, 0, 2)).reshape(4 * D, P), bias_prep],
        axis=0)

    out_shapes = (
        jax.ShapeDtypeStruct((B, H), jnp.float32),
        jax.ShapeDtypeStruct((1, 1), jnp.float32),
        jax.ShapeDtypeStruct((B, D), jnp.float32),
    )
    hout, loss, lpre = pl.pallas_call(
        _seq_gru_kernel,
        out_shape=out_shapes,
        out_specs=(
            pl.BlockSpec(memory_space=pltpu.VMEM),
            pl.BlockSpec(memory_space=pltpu.SMEM),
            pl.BlockSpec(memory_space=pltpu.VMEM),
        ),
        scratch_shapes=[pltpu.VMEM((B, H), jnp.float32)],
    )(hidden0, Xm, Fm, X, M,
      Whw, bhw, W2, b2.reshape(1, -1),
      W_ih.T, b_ih.reshape(1, -1), Wp)

    h2 = h.at[i_obs].set(hout)
    return (h2, loss[0, 0], lpre)
